# Initial kernel scaffold; baseline (speedup 1.0000x reference)
import functools

import jax
import jax.numpy as jnp
from jax import lax
from jax.experimental import pallas as pl
from jax.experimental.pallas import tpu as pltpu
from jax.experimental.pallas import tpu_sc as plsc

N = 10000
D = 128
E = 320000
NUM_LAYERS = 14

NC = 2
NS = 16
NW = NC * NS
L = 16

ECHUNK = 128
NCH = 79
E_PAD = NW * NCH * ECHUNK
AGG_ROWS = 10240
ZROWS = AGG_ROWS // NS
RW = N // NS

_mesh = plsc.VectorSubcoreMesh(core_axis_name="c", subcore_axis_name="s")


@functools.partial(
    pl.kernel,
    out_type=jax.ShapeDtypeStruct((NC, N, D), jnp.float32),
    mesh=_mesh,
    scratch_types=[
        pltpu.VMEM((NCH, ECHUNK), jnp.int32),
        pltpu.VMEM((NCH, ECHUNK), jnp.int32),
        pltpu.VMEM((ECHUNK, D), jnp.float32),
        pltpu.VMEM_SHARED((AGG_ROWS, D), jnp.float32),
        pltpu.SemaphoreType.DMA,
    ],
)
def _segsum(src_hbm, dst_hbm, x_hbm, out_hbm, src_v, dst_v, rows_v, agg_sh, sem):
    c = lax.axis_index("c")
    s = lax.axis_index("s")
    wid = s * NC + c

    pltpu.sync_copy(src_hbm.at[wid], src_v)
    pltpu.sync_copy(dst_hbm.at[wid], dst_v)

    zero = jnp.zeros((L,), jnp.float32)

    def _zrow(i, carry):
        for k in range(D // L):
            rows_v[i, pl.ds(k * L, L)] = zero
        return carry

    lax.fori_loop(0, ECHUNK, _zrow, 0)
    for z in range(ZROWS // ECHUNK):
        pltpu.sync_copy(rows_v, agg_sh.at[pl.ds(s * ZROWS + z * ECHUNK, ECHUNK)])
    plsc.subcore_barrier()

    def _body(j, carry):
        pltpu.async_copy(x_hbm.at[src_v.at[j]], rows_v, sem).wait()
        pltpu.sync_copy(rows_v, agg_sh.at[dst_v.at[j]], add=True)
        return carry

    lax.fori_loop(0, NCH, _body, 0)
    plsc.subcore_barrier()

    pltpu.sync_copy(agg_sh.at[pl.ds(s * RW, RW)], out_hbm.at[c].at[pl.ds(s * RW, RW)])


BN = 2000


def _gcn_body(a_ref, x_ref, wn_ref, ws_ref, b_ref, o_ref):
    agg = a_ref[0] + a_ref[1]
    h = jnp.dot(agg, wn_ref[...], preferred_element_type=jnp.float32)
    h = h + jnp.dot(x_ref[...], ws_ref[...], preferred_element_type=jnp.float32)
    o_ref[...] = jnp.maximum(h + b_ref[...], 0.0)


def _gcn_res_body(a_ref, x_ref, t_ref, wn_ref, ws_ref, b_ref, o_ref):
    agg = a_ref[0] + a_ref[1]
    h = jnp.dot(agg, wn_ref[...], preferred_element_type=jnp.float32)
    h = h + jnp.dot(x_ref[...], ws_ref[...], preferred_element_type=jnp.float32)
    o_ref[...] = (t_ref[...] + jnp.maximum(h + b_ref[...], 0.0)) * 0.5


def _gcn_final_body(a_ref, x_ref, wn_ref, ws_ref, b_ref, we_ref, be_ref, h_ref, c_ref):
    agg = a_ref[0] + a_ref[1]
    h = jnp.dot(agg, wn_ref[...], preferred_element_type=jnp.float32)
    h = h + jnp.dot(x_ref[...], ws_ref[...], preferred_element_type=jnp.float32)
    h = jnp.maximum(h + b_ref[...], 0.0)
    h_ref[...] = h
    c_ref[...] = jnp.dot(h, we_ref[...], preferred_element_type=jnp.float32) + be_ref[...]


_a_spec = pl.BlockSpec((NC, BN, D), lambda i: (0, i, 0))
_x_spec = pl.BlockSpec((BN, D), lambda i: (i, 0))
_w_spec = pl.BlockSpec((D, D), lambda i: (0, 0))
_b_spec = pl.BlockSpec((1, D), lambda i: (0, 0))
_o_spec = pl.BlockSpec((BN, D), lambda i: (i, 0))
_GRID = (N // BN,)
_f32 = jnp.float32

_gcn = pl.pallas_call(
    _gcn_body,
    grid=_GRID,
    in_specs=[_a_spec, _x_spec, _w_spec, _w_spec, _b_spec],
    out_specs=_o_spec,
    out_shape=jax.ShapeDtypeStruct((N, D), _f32),
)

_gcn_res = pl.pallas_call(
    _gcn_res_body,
    grid=_GRID,
    in_specs=[_a_spec, _x_spec, _x_spec, _w_spec, _w_spec, _b_spec],
    out_specs=_o_spec,
    out_shape=jax.ShapeDtypeStruct((N, D), _f32),
)

_gcn_final = pl.pallas_call(
    _gcn_final_body,
    grid=_GRID,
    in_specs=[_a_spec, _x_spec, _w_spec, _w_spec, _b_spec, _w_spec, _b_spec],
    out_specs=[_o_spec, _o_spec],
    out_shape=[
        jax.ShapeDtypeStruct((N, D), _f32),
        jax.ShapeDtypeStruct((N, D), _f32),
    ],
)


def kernel(neighbours, shape_features, Wn, Ws, b, We, be):
    src = neighbours[0]
    dst = neighbours[1]
    pad = E_PAD - E
    src_p = jnp.concatenate([src, jnp.zeros((pad,), jnp.int32)]).reshape(NW, NCH, ECHUNK)
    dst_p = jnp.concatenate([dst, jnp.full((pad,), N, jnp.int32)]).reshape(NW, NCH, ECHUNK)
    We_pad = jnp.pad(We, ((0, 0), (0, D - 3)))
    be_pad = jnp.pad(be, (0, D - 3)).reshape(1, D)
    b2 = b.reshape(NUM_LAYERS, 1, D)

    x = shape_features
    h = _gcn(_segsum(src_p, dst_p, x), x, Wn[0], Ws[0], b2[0])
    li = 1
    for _ in range(6):
        temp = h
        h = _gcn(_segsum(src_p, dst_p, h), h, Wn[li], Ws[li], b2[li])
        li += 1
        h = _gcn_res(_segsum(src_p, dst_p, h), h, temp, Wn[li], Ws[li], b2[li])
        li += 1
    h, coords_pad = _gcn_final(
        _segsum(src_p, dst_p, h), h, Wn[13], Ws[13], b2[13], We_pad, be_pad
    )
    return (h, coords_pad[:, :3])


# speedup vs baseline: 1.7101x; 1.7101x over previous
import functools

import jax
import jax.numpy as jnp
from jax import lax
from jax.experimental import pallas as pl
from jax.experimental.pallas import tpu as pltpu
from jax.experimental.pallas import tpu_sc as plsc

N = 10000
D = 128
E = 320000
NUM_LAYERS = 14

NC = 2
NS = 16
NW = NC * NS
L = 16

ECHUNK = 128
NCH = 80
GB = 2
GBI = 4
E_PAD = NW * NCH * ECHUNK
AGG_ROWS = 10112
ZROWS = AGG_ROWS // NS

_mesh = plsc.VectorSubcoreMesh(core_axis_name="c", subcore_axis_name="s")


@functools.partial(
    pl.kernel,
    out_type=jax.ShapeDtypeStruct((NC, AGG_ROWS, D), jnp.float32),
    mesh=_mesh,
    scratch_types=[
        pltpu.VMEM((GBI, 2, ECHUNK), jnp.int32),
        pltpu.VMEM((ECHUNK, D), jnp.float32),
        pltpu.VMEM((ECHUNK, D), jnp.float32),
        pltpu.VMEM_SHARED((AGG_ROWS, D), jnp.float32),
        pltpu.SemaphoreType.DMA,
        pltpu.SemaphoreType.DMA,
        pltpu.SemaphoreType.DMA,
        pltpu.SemaphoreType.DMA,
        pltpu.SemaphoreType.DMA,
        pltpu.SemaphoreType.DMA,
    ],
)
def _segsum(edges_hbm, x_hbm, out_hbm, idx_v, r0, r1, agg_sh, *sems):
    rows = (r0, r1)
    sem_g = sems[:GB]
    sem_i = sems[GB:]
    c = lax.axis_index("c")
    s = lax.axis_index("s")
    wid = s * NC + c
    edges_w = edges_hbm.at[wid]

    for i in range(GBI):
        pltpu.async_copy(edges_w.at[i], idx_v.at[i], sem_i[i])

    zero = jnp.zeros((L,), jnp.float32)

    def _zrow(i, carry):
        for k in range(D // L):
            r0[i, pl.ds(k * L, L)] = zero
        return carry

    lax.fori_loop(0, ECHUNK, _zrow, 0)
    for z in range(ZROWS // ECHUNK):
        pltpu.sync_copy(r0, agg_sh.at[pl.ds(s * ZROWS + z * ECHUNK, ECHUNK)])
    zrem = ZROWS % ECHUNK
    if zrem:
        pltpu.sync_copy(
            r0.at[pl.ds(0, zrem)],
            agg_sh.at[pl.ds(s * ZROWS + (ZROWS // ECHUNK) * ECHUNK, zrem)],
        )
    plsc.subcore_barrier()

    for g in range(GB):
        pltpu.make_async_copy(edges_w.at[g], idx_v.at[g], sem_i[g]).wait()
        pltpu.async_copy(x_hbm.at[idx_v.at[g].at[0]], rows[g], sem_g[g])

    def _body(jj, carry):
        j0 = jj * GBI
        for g in range(GBI):
            b = g % GB
            inext = (g + GB) % GBI
            pltpu.make_async_copy(x_hbm.at[idx_v.at[g].at[0]], rows[b], sem_g[b]).wait()
            pltpu.sync_copy(rows[b], agg_sh.at[idx_v.at[g].at[1]], add=True)
            pltpu.async_copy(edges_w.at[j0 + g + GBI], idx_v.at[g], sem_i[g])
            pltpu.make_async_copy(
                edges_w.at[j0 + g + GB], idx_v.at[inext], sem_i[inext]
            ).wait()
            pltpu.async_copy(x_hbm.at[idx_v.at[inext].at[0]], rows[b], sem_g[b])
        return carry

    lax.fori_loop(0, NCH // GBI, _body, 0)
    for g in range(GB):
        pltpu.make_async_copy(x_hbm.at[idx_v.at[g].at[0]], rows[g], sem_g[g]).wait()
    for i in range(GB, GBI):
        pltpu.make_async_copy(edges_w.at[NCH + i], idx_v.at[i], sem_i[i]).wait()
    plsc.subcore_barrier()

    pltpu.sync_copy(
        agg_sh.at[pl.ds(s * ZROWS, ZROWS)], out_hbm.at[c].at[pl.ds(s * ZROWS, ZROWS)]
    )


BN = 2000


def _gcn_body(a_ref, x_ref, wn_ref, ws_ref, b_ref, o_ref):
    agg = a_ref[0] + a_ref[1]
    h = jnp.dot(agg, wn_ref[...], preferred_element_type=jnp.float32)
    h = h + jnp.dot(x_ref[...], ws_ref[...], preferred_element_type=jnp.float32)
    o_ref[...] = jnp.maximum(h + b_ref[...], 0.0)


def _gcn_res_body(a_ref, x_ref, t_ref, wn_ref, ws_ref, b_ref, o_ref):
    agg = a_ref[0] + a_ref[1]
    h = jnp.dot(agg, wn_ref[...], preferred_element_type=jnp.float32)
    h = h + jnp.dot(x_ref[...], ws_ref[...], preferred_element_type=jnp.float32)
    o_ref[...] = (t_ref[...] + jnp.maximum(h + b_ref[...], 0.0)) * 0.5


def _gcn_final_body(a_ref, x_ref, wn_ref, ws_ref, b_ref, we_ref, be_ref, h_ref, c_ref):
    agg = a_ref[0] + a_ref[1]
    h = jnp.dot(agg, wn_ref[...], preferred_element_type=jnp.float32)
    h = h + jnp.dot(x_ref[...], ws_ref[...], preferred_element_type=jnp.float32)
    h = jnp.maximum(h + b_ref[...], 0.0)
    h_ref[...] = h
    c_ref[...] = jnp.dot(h, we_ref[...], preferred_element_type=jnp.float32) + be_ref[...]


_a_spec = pl.BlockSpec((NC, BN, D), lambda i: (0, i, 0))
_x_spec = pl.BlockSpec((BN, D), lambda i: (i, 0))
_w_spec = pl.BlockSpec((D, D), lambda i: (0, 0))
_b_spec = pl.BlockSpec((1, D), lambda i: (0, 0))
_o_spec = pl.BlockSpec((BN, D), lambda i: (i, 0))
_GRID = (N // BN,)
_f32 = jnp.float32

_gcn = pl.pallas_call(
    _gcn_body,
    grid=_GRID,
    in_specs=[_a_spec, _x_spec, _w_spec, _w_spec, _b_spec],
    out_specs=_o_spec,
    out_shape=jax.ShapeDtypeStruct((N, D), _f32),
)

_gcn_res = pl.pallas_call(
    _gcn_res_body,
    grid=_GRID,
    in_specs=[_a_spec, _x_spec, _x_spec, _w_spec, _w_spec, _b_spec],
    out_specs=_o_spec,
    out_shape=jax.ShapeDtypeStruct((N, D), _f32),
)

_gcn_final = pl.pallas_call(
    _gcn_final_body,
    grid=_GRID,
    in_specs=[_a_spec, _x_spec, _w_spec, _w_spec, _b_spec, _w_spec, _b_spec],
    out_specs=[_o_spec, _o_spec],
    out_shape=[
        jax.ShapeDtypeStruct((N, D), _f32),
        jax.ShapeDtypeStruct((N, D), _f32),
    ],
)


def kernel(neighbours, shape_features, Wn, Ws, b, We, be):
    src = neighbours[0]
    dst = neighbours[1]
    pad = E_PAD - E
    src_p = jnp.concatenate([src, jnp.zeros((pad,), jnp.int32)]).reshape(NW, NCH, ECHUNK)
    dst_p = jnp.concatenate([dst, jnp.full((pad,), N, jnp.int32)]).reshape(NW, NCH, ECHUNK)
    edges = jnp.stack([src_p, dst_p], axis=2)
    edges = jnp.concatenate(
        [edges, jnp.zeros((NW, GBI, 2, ECHUNK), jnp.int32)], axis=1
    )
    We_pad = jnp.pad(We, ((0, 0), (0, D - 3)))
    be_pad = jnp.pad(be, (0, D - 3)).reshape(1, D)
    b2 = b.reshape(NUM_LAYERS, 1, D)

    x = shape_features
    h = _gcn(_segsum(edges, x), x, Wn[0], Ws[0], b2[0])
    li = 1
    for _ in range(6):
        temp = h
        h = _gcn(_segsum(edges, h), h, Wn[li], Ws[li], b2[li])
        li += 1
        h = _gcn_res(_segsum(edges, h), h, temp, Wn[li], Ws[li], b2[li])
        li += 1
    h, coords_pad = _gcn_final(
        _segsum(edges, h), h, Wn[13], Ws[13], b2[13], We_pad, be_pad
    )
    return (h, coords_pad[:, :3])


# speedup vs baseline: 1.7873x; 1.0452x over previous
import functools

import jax
import jax.numpy as jnp
from jax import lax
from jax.experimental import pallas as pl
from jax.experimental.pallas import tpu as pltpu
from jax.experimental.pallas import tpu_sc as plsc

N = 10000
D = 128
E = 320000
NUM_LAYERS = 14

NC = 2
NS = 16
NW = NC * NS
L = 16

ECHUNK = 128
BCH = 8
NCH = 80
NBLK = NCH // BCH
GB = 2
E_PAD = NW * NCH * ECHUNK
AGG_ROWS = 10112
ZROWS = AGG_ROWS // NS

_mesh = plsc.VectorSubcoreMesh(core_axis_name="c", subcore_axis_name="s")


@functools.partial(
    pl.kernel,
    out_type=jax.ShapeDtypeStruct((NC, AGG_ROWS, D), jnp.float32),
    mesh=_mesh,
    scratch_types=[
        pltpu.VMEM((4 * BCH, ECHUNK), jnp.int32),
        pltpu.VMEM((ECHUNK, D), jnp.float32),
        pltpu.VMEM((ECHUNK, D), jnp.float32),
        pltpu.VMEM_SHARED((AGG_ROWS, D), jnp.float32),
        pltpu.SemaphoreType.DMA,
        pltpu.SemaphoreType.DMA,
        pltpu.SemaphoreType.DMA,
    ],
)
def _segsum(edges_hbm, x_hbm, out_hbm, islab, r0, r1, agg_sh, sem_g0, sem_g1, sem_i):
    rows = (r0, r1)
    sem_g = (sem_g0, sem_g1)
    c = lax.axis_index("c")
    s = lax.axis_index("s")
    wid = s * NC + c
    edges_w = edges_hbm.at[wid]

    pltpu.async_copy(edges_w.at[0], islab.at[pl.ds(0, 2 * BCH)], sem_i)

    zero = jnp.zeros((L,), jnp.float32)

    def _zrow(i, carry):
        for k in range(D // L):
            r0[i, pl.ds(k * L, L)] = zero
        return carry

    lax.fori_loop(0, ECHUNK, _zrow, 0)
    for z in range(ZROWS // ECHUNK):
        pltpu.sync_copy(r0, agg_sh.at[pl.ds(s * ZROWS + z * ECHUNK, ECHUNK)])
    zrem = ZROWS % ECHUNK
    if zrem:
        pltpu.sync_copy(
            r0.at[pl.ds(0, zrem)],
            agg_sh.at[pl.ds(s * ZROWS + (ZROWS // ECHUNK) * ECHUNK, zrem)],
        )
    pltpu.make_async_copy(edges_w.at[0], islab.at[pl.ds(0, 2 * BCH)], sem_i).wait()
    pltpu.async_copy(edges_w.at[1], islab.at[pl.ds(2 * BCH, 2 * BCH)], sem_i)
    plsc.subcore_barrier()

    for g in range(GB):
        pltpu.async_copy(x_hbm.at[islab.at[g]], rows[g], sem_g[g])

    def _body(B, carry):
        pbase = lax.rem(B, 2) * (2 * BCH)
        qbase = (2 * BCH) - pbase
        for k in range(BCH):
            b = k % GB
            pltpu.make_async_copy(x_hbm.at[islab.at[pbase + k]], rows[b], sem_g[b]).wait()
            pltpu.sync_copy(rows[b], agg_sh.at[islab.at[pbase + BCH + k]], add=True)
            if k == BCH - 2:
                pltpu.make_async_copy(
                    edges_w.at[B + 1], islab.at[pl.ds(qbase, 2 * BCH)], sem_i
                ).wait()
            if k < BCH - 2:
                idx_row = islab.at[pbase + k + GB]
            else:
                idx_row = islab.at[qbase + (k - (BCH - 2))]
            pltpu.async_copy(x_hbm.at[idx_row], rows[b], sem_g[b])
            if k == BCH - 1:
                pltpu.async_copy(
                    edges_w.at[B + 2], islab.at[pl.ds(pbase, 2 * BCH)], sem_i
                )
        return carry

    lax.fori_loop(0, NBLK, _body, 0)
    for g in range(GB):
        pltpu.make_async_copy(x_hbm.at[islab.at[g]], rows[g], sem_g[g]).wait()
    pltpu.make_async_copy(
        edges_w.at[NBLK + 1], islab.at[pl.ds(2 * BCH, 2 * BCH)], sem_i
    ).wait()
    plsc.subcore_barrier()

    pltpu.sync_copy(
        agg_sh.at[pl.ds(s * ZROWS, ZROWS)], out_hbm.at[c].at[pl.ds(s * ZROWS, ZROWS)]
    )


BN = 2000


def _gcn_body(a_ref, x_ref, wn_ref, ws_ref, b_ref, o_ref):
    agg = a_ref[0] + a_ref[1]
    h = jnp.dot(agg, wn_ref[...], preferred_element_type=jnp.float32)
    h = h + jnp.dot(x_ref[...], ws_ref[...], preferred_element_type=jnp.float32)
    o_ref[...] = jnp.maximum(h + b_ref[...], 0.0)


def _gcn_res_body(a_ref, x_ref, t_ref, wn_ref, ws_ref, b_ref, o_ref):
    agg = a_ref[0] + a_ref[1]
    h = jnp.dot(agg, wn_ref[...], preferred_element_type=jnp.float32)
    h = h + jnp.dot(x_ref[...], ws_ref[...], preferred_element_type=jnp.float32)
    o_ref[...] = (t_ref[...] + jnp.maximum(h + b_ref[...], 0.0)) * 0.5


def _gcn_final_body(a_ref, x_ref, wn_ref, ws_ref, b_ref, we_ref, be_ref, h_ref, c_ref):
    agg = a_ref[0] + a_ref[1]
    h = jnp.dot(agg, wn_ref[...], preferred_element_type=jnp.float32)
    h = h + jnp.dot(x_ref[...], ws_ref[...], preferred_element_type=jnp.float32)
    h = jnp.maximum(h + b_ref[...], 0.0)
    h_ref[...] = h
    c_ref[...] = jnp.dot(h, we_ref[...], preferred_element_type=jnp.float32) + be_ref[...]


_a_spec = pl.BlockSpec((NC, BN, D), lambda i: (0, i, 0))
_x_spec = pl.BlockSpec((BN, D), lambda i: (i, 0))
_w_spec = pl.BlockSpec((D, D), lambda i: (0, 0))
_b_spec = pl.BlockSpec((1, D), lambda i: (0, 0))
_o_spec = pl.BlockSpec((BN, D), lambda i: (i, 0))
_GRID = (N // BN,)
_f32 = jnp.float32

_gcn = pl.pallas_call(
    _gcn_body,
    grid=_GRID,
    in_specs=[_a_spec, _x_spec, _w_spec, _w_spec, _b_spec],
    out_specs=_o_spec,
    out_shape=jax.ShapeDtypeStruct((N, D), _f32),
)

_gcn_res = pl.pallas_call(
    _gcn_res_body,
    grid=_GRID,
    in_specs=[_a_spec, _x_spec, _x_spec, _w_spec, _w_spec, _b_spec],
    out_specs=_o_spec,
    out_shape=jax.ShapeDtypeStruct((N, D), _f32),
)

_gcn_final = pl.pallas_call(
    _gcn_final_body,
    grid=_GRID,
    in_specs=[_a_spec, _x_spec, _w_spec, _w_spec, _b_spec, _w_spec, _b_spec],
    out_specs=[_o_spec, _o_spec],
    out_shape=[
        jax.ShapeDtypeStruct((N, D), _f32),
        jax.ShapeDtypeStruct((N, D), _f32),
    ],
)


def kernel(neighbours, shape_features, Wn, Ws, b, We, be):
    src = neighbours[0]
    dst = neighbours[1]
    pad = E_PAD - E
    src_p = jnp.concatenate([src, jnp.zeros((pad,), jnp.int32)]).reshape(NW, NCH, ECHUNK)
    dst_p = jnp.concatenate([dst, jnp.full((pad,), N, jnp.int32)]).reshape(NW, NCH, ECHUNK)
    edges = jnp.concatenate(
        [
            src_p.reshape(NW, NBLK, BCH, ECHUNK),
            dst_p.reshape(NW, NBLK, BCH, ECHUNK),
        ],
        axis=2,
    )
    edges = jnp.concatenate(
        [edges, jnp.zeros((NW, 2, 2 * BCH, ECHUNK), jnp.int32)], axis=1
    )
    We_pad = jnp.pad(We, ((0, 0), (0, D - 3)))
    be_pad = jnp.pad(be, (0, D - 3)).reshape(1, D)
    b2 = b.reshape(NUM_LAYERS, 1, D)

    x = shape_features
    h = _gcn(_segsum(edges, x), x, Wn[0], Ws[0], b2[0])
    li = 1
    for _ in range(6):
        temp = h
        h = _gcn(_segsum(edges, h), h, Wn[li], Ws[li], b2[li])
        li += 1
        h = _gcn_res(_segsum(edges, h), h, temp, Wn[li], Ws[li], b2[li])
        li += 1
    h, coords_pad = _gcn_final(
        _segsum(edges, h), h, Wn[13], Ws[13], b2[13], We_pad, be_pad
    )
    return (h, coords_pad[:, :3])


# speedup vs baseline: 2.6951x; 1.5079x over previous
import functools

import jax
import jax.numpy as jnp
from jax import lax
from jax.experimental import pallas as pl
from jax.experimental.pallas import tpu as pltpu
from jax.experimental.pallas import tpu_sc as plsc

N = 10000
D = 128
E = 320000
NUM_LAYERS = 14

NC = 2
NS = 16
NW = NC * NS
L = 16

ECHUNK = 128
BCH = 8
NCH = 80
NBLK = NCH // BCH
GB = 2
E_PAD = NW * NCH * ECHUNK
AGG_ROWS = 10112
ZROWS = AGG_ROWS // NS

_mesh = plsc.VectorSubcoreMesh(core_axis_name="c", subcore_axis_name="s")


@functools.partial(
    pl.kernel,
    out_type=jax.ShapeDtypeStruct((NC, AGG_ROWS, D), jnp.float32),
    mesh=_mesh,
    scratch_types=[
        pltpu.VMEM((NCH, ECHUNK), jnp.int32),
        pltpu.VMEM((NCH, ECHUNK), jnp.int32),
        pltpu.VMEM((ECHUNK, D), jnp.float32),
        pltpu.VMEM_SHARED((AGG_ROWS, D), jnp.float32),
        pltpu.SemaphoreType.DMA,
    ],
)
def _segsum(src_hbm, dst_hbm, x_hbm, out_hbm, src_v, dst_v, r0, agg_sh, sem_g):
    c = lax.axis_index("c")
    s = lax.axis_index("s")
    wid = s * NC + c

    pltpu.sync_copy(src_hbm.at[wid], src_v)
    pltpu.sync_copy(dst_hbm.at[wid], dst_v)

    zero = jnp.zeros((L,), jnp.float32)

    def _zrow(i, carry):
        for k in range(D // L):
            r0[i, pl.ds(k * L, L)] = zero
        return carry

    lax.fori_loop(0, ECHUNK, _zrow, 0)
    for z in range(ZROWS // ECHUNK):
        pltpu.sync_copy(r0, agg_sh.at[pl.ds(s * ZROWS + z * ECHUNK, ECHUNK)])
    zrem = ZROWS % ECHUNK
    if zrem:
        pltpu.sync_copy(
            r0.at[pl.ds(0, zrem)],
            agg_sh.at[pl.ds(s * ZROWS + (ZROWS // ECHUNK) * ECHUNK, zrem)],
        )
    plsc.subcore_barrier()

    def _body(j, carry):
        pltpu.async_copy(x_hbm.at[src_v.at[j]], r0, sem_g).wait()
        pltpu.sync_copy(r0, agg_sh.at[dst_v.at[j]], add=True)
        return carry

    lax.fori_loop(0, NCH, _body, 0)
    plsc.subcore_barrier()

    pltpu.sync_copy(
        agg_sh.at[pl.ds(s * ZROWS, ZROWS)], out_hbm.at[c].at[pl.ds(s * ZROWS, ZROWS)]
    )


BN = 2000


def _gcn_body(a_ref, x_ref, wn_ref, ws_ref, b_ref, o_ref):
    agg = a_ref[0] + a_ref[1]
    h = jnp.dot(agg, wn_ref[...], preferred_element_type=jnp.float32)
    h = h + jnp.dot(x_ref[...], ws_ref[...], preferred_element_type=jnp.float32)
    o_ref[...] = jnp.maximum(h + b_ref[...], 0.0)


def _gcn_res_body(a_ref, x_ref, t_ref, wn_ref, ws_ref, b_ref, o_ref):
    agg = a_ref[0] + a_ref[1]
    h = jnp.dot(agg, wn_ref[...], preferred_element_type=jnp.float32)
    h = h + jnp.dot(x_ref[...], ws_ref[...], preferred_element_type=jnp.float32)
    o_ref[...] = (t_ref[...] + jnp.maximum(h + b_ref[...], 0.0)) * 0.5


def _gcn_final_body(a_ref, x_ref, wn_ref, ws_ref, b_ref, we_ref, be_ref, h_ref, c_ref):
    agg = a_ref[0] + a_ref[1]
    h = jnp.dot(agg, wn_ref[...], preferred_element_type=jnp.float32)
    h = h + jnp.dot(x_ref[...], ws_ref[...], preferred_element_type=jnp.float32)
    h = jnp.maximum(h + b_ref[...], 0.0)
    h_ref[...] = h
    c_ref[...] = jnp.dot(h, we_ref[...], preferred_element_type=jnp.float32) + be_ref[...]


_a_spec = pl.BlockSpec((NC, BN, D), lambda i: (0, i, 0))
_x_spec = pl.BlockSpec((BN, D), lambda i: (i, 0))
_w_spec = pl.BlockSpec((D, D), lambda i: (0, 0))
_b_spec = pl.BlockSpec((1, D), lambda i: (0, 0))
_o_spec = pl.BlockSpec((BN, D), lambda i: (i, 0))
_GRID = (N // BN,)
_f32 = jnp.float32

_gcn = pl.pallas_call(
    _gcn_body,
    grid=_GRID,
    in_specs=[_a_spec, _x_spec, _w_spec, _w_spec, _b_spec],
    out_specs=_o_spec,
    out_shape=jax.ShapeDtypeStruct((N, D), _f32),
)

_gcn_res = pl.pallas_call(
    _gcn_res_body,
    grid=_GRID,
    in_specs=[_a_spec, _x_spec, _x_spec, _w_spec, _w_spec, _b_spec],
    out_specs=_o_spec,
    out_shape=jax.ShapeDtypeStruct((N, D), _f32),
)

_gcn_final = pl.pallas_call(
    _gcn_final_body,
    grid=_GRID,
    in_specs=[_a_spec, _x_spec, _w_spec, _w_spec, _b_spec, _w_spec, _b_spec],
    out_specs=[_o_spec, _o_spec],
    out_shape=[
        jax.ShapeDtypeStruct((N, D), _f32),
        jax.ShapeDtypeStruct((N, D), _f32),
    ],
)


def kernel(neighbours, shape_features, Wn, Ws, b, We, be):
    src = neighbours[0]
    dst = neighbours[1]
    pad = E_PAD - E
    src_p = jnp.concatenate([src, jnp.zeros((pad,), jnp.int32)]).reshape(NW, NCH, ECHUNK)
    dst_p = jnp.concatenate([dst, jnp.full((pad,), N, jnp.int32)]).reshape(NW, NCH, ECHUNK)
    We_pad = jnp.pad(We, ((0, 0), (0, D - 3)))
    be_pad = jnp.pad(be, (0, D - 3)).reshape(1, D)
    b2 = b.reshape(NUM_LAYERS, 1, D)

    x = shape_features
    h = _gcn(_segsum(src_p, dst_p, x), x, Wn[0], Ws[0], b2[0])
    li = 1
    for _ in range(6):
        temp = h
        h = _gcn(_segsum(src_p, dst_p, h), h, Wn[li], Ws[li], b2[li])
        li += 1
        h = _gcn_res(_segsum(src_p, dst_p, h), h, temp, Wn[li], Ws[li], b2[li])
        li += 1
    h, coords_pad = _gcn_final(
        _segsum(src_p, dst_p, h), h, Wn[13], Ws[13], b2[13], We_pad, be_pad
    )
    return (h, coords_pad[:, :3])


# speedup vs baseline: 3.9706x; 1.4732x over previous
import functools

import jax
import jax.numpy as jnp
from jax import lax
from jax.experimental import pallas as pl
from jax.experimental.pallas import tpu as pltpu
from jax.experimental.pallas import tpu_sc as plsc

N = 10000
D = 128
E = 320000
NUM_LAYERS = 14

NC = 2
NS = 16
NW = NC * NS
L = 16

ECHUNK = 128
NCH = 79
E_PAD = NW * NCH * ECHUNK
AGG_ROWS = 10240
ZROWS = AGG_ROWS // NS

_mesh = plsc.VectorSubcoreMesh(core_axis_name="c", subcore_axis_name="s")


@functools.partial(
    pl.kernel,
    out_type=jax.ShapeDtypeStruct((NC, AGG_ROWS, D), jnp.float32),
    mesh=_mesh,
    scratch_types=[
        pltpu.VMEM((NCH, ECHUNK), jnp.int32),
        pltpu.VMEM((NCH, ECHUNK), jnp.int32),
        pltpu.VMEM((ECHUNK, D), jnp.float32),
        pltpu.VMEM_SHARED((AGG_ROWS, D), jnp.float32),
        pltpu.SemaphoreType.DMA,
    ],
)
def _segsum(src_hbm, dst_hbm, x_hbm, out_hbm, src_v, dst_v, r0, agg_sh, sem_g):
    c = lax.axis_index("c")
    s = lax.axis_index("s")
    wid = s * NC + c

    pltpu.sync_copy(src_hbm.at[wid], src_v)
    pltpu.sync_copy(dst_hbm.at[wid], dst_v)

    zero = jnp.zeros((L,), jnp.float32)

    def _zrow(i, carry):
        for k in range(D // L):
            r0[i, pl.ds(k * L, L)] = zero
        return carry

    lax.fori_loop(0, ECHUNK, _zrow, 0)
    for z in range(ZROWS // ECHUNK):
        pltpu.sync_copy(r0, agg_sh.at[pl.ds(s * ZROWS + z * ECHUNK, ECHUNK)])
    zrem = ZROWS % ECHUNK
    if zrem:
        pltpu.sync_copy(
            r0.at[pl.ds(0, zrem)],
            agg_sh.at[pl.ds(s * ZROWS + (ZROWS // ECHUNK) * ECHUNK, zrem)],
        )
    plsc.subcore_barrier()

    def _body(j, carry):
        pltpu.async_copy(x_hbm.at[src_v.at[j]], r0, sem_g).wait()
        pltpu.sync_copy(r0, agg_sh.at[dst_v.at[j]], add=True)
        return carry

    lax.fori_loop(0, NCH, _body, 0)
    plsc.subcore_barrier()

    pltpu.sync_copy(
        agg_sh.at[pl.ds(s * ZROWS, ZROWS)], out_hbm.at[c].at[pl.ds(s * ZROWS, ZROWS)]
    )


BN = 2000


def _gcn_body(a_ref, x_ref, wn_ref, ws_ref, b_ref, o_ref):
    agg = a_ref[0] + a_ref[1]
    h = jnp.dot(agg, wn_ref[...], preferred_element_type=jnp.float32)
    h = h + jnp.dot(x_ref[...], ws_ref[...], preferred_element_type=jnp.float32)
    o_ref[...] = jnp.maximum(h + b_ref[...], 0.0)


def _gcn_res_body(a_ref, x_ref, t_ref, wn_ref, ws_ref, b_ref, o_ref):
    agg = a_ref[0] + a_ref[1]
    h = jnp.dot(agg, wn_ref[...], preferred_element_type=jnp.float32)
    h = h + jnp.dot(x_ref[...], ws_ref[...], preferred_element_type=jnp.float32)
    o_ref[...] = (t_ref[...] + jnp.maximum(h + b_ref[...], 0.0)) * 0.5


def _gcn_final_body(a_ref, x_ref, wn_ref, ws_ref, b_ref, we_ref, be_ref, h_ref, c_ref):
    agg = a_ref[0] + a_ref[1]
    h = jnp.dot(agg, wn_ref[...], preferred_element_type=jnp.float32)
    h = h + jnp.dot(x_ref[...], ws_ref[...], preferred_element_type=jnp.float32)
    h = jnp.maximum(h + b_ref[...], 0.0)
    h_ref[...] = h
    c_ref[...] = jnp.dot(h, we_ref[...], preferred_element_type=jnp.float32) + be_ref[...]


_a_spec = pl.BlockSpec((NC, BN, D), lambda i: (0, i, 0))
_x_spec = pl.BlockSpec((BN, D), lambda i: (i, 0))
_w_spec = pl.BlockSpec((D, D), lambda i: (0, 0))
_b_spec = pl.BlockSpec((1, D), lambda i: (0, 0))
_o_spec = pl.BlockSpec((BN, D), lambda i: (i, 0))
_GRID = (N // BN,)
_f32 = jnp.float32

_gcn = pl.pallas_call(
    _gcn_body,
    grid=_GRID,
    in_specs=[_a_spec, _x_spec, _w_spec, _w_spec, _b_spec],
    out_specs=_o_spec,
    out_shape=jax.ShapeDtypeStruct((N, D), _f32),
)

_gcn_res = pl.pallas_call(
    _gcn_res_body,
    grid=_GRID,
    in_specs=[_a_spec, _x_spec, _x_spec, _w_spec, _w_spec, _b_spec],
    out_specs=_o_spec,
    out_shape=jax.ShapeDtypeStruct((N, D), _f32),
)

_gcn_final = pl.pallas_call(
    _gcn_final_body,
    grid=_GRID,
    in_specs=[_a_spec, _x_spec, _w_spec, _w_spec, _b_spec, _w_spec, _b_spec],
    out_specs=[_o_spec, _o_spec],
    out_shape=[
        jax.ShapeDtypeStruct((N, D), _f32),
        jax.ShapeDtypeStruct((N, D), _f32),
    ],
)


def kernel(neighbours, shape_features, Wn, Ws, b, We, be):
    src = neighbours[0]
    dst = neighbours[1]
    pad = E_PAD - E
    src_p = jnp.concatenate([src, jnp.zeros((pad,), jnp.int32)]).reshape(NW, NCH, ECHUNK)
    dst_p = jnp.concatenate([dst, jnp.full((pad,), N, jnp.int32)]).reshape(NW, NCH, ECHUNK)
    We_pad = jnp.pad(We, ((0, 0), (0, D - 3)))
    be_pad = jnp.pad(be, (0, D - 3)).reshape(1, D)
    b2 = b.reshape(NUM_LAYERS, 1, D)

    x = shape_features
    h = _gcn(_segsum(src_p, dst_p, x), x, Wn[0], Ws[0], b2[0])
    li = 1
    for _ in range(6):
        temp = h
        h = _gcn(_segsum(src_p, dst_p, h), h, Wn[li], Ws[li], b2[li])
        li += 1
        h = _gcn_res(_segsum(src_p, dst_p, h), h, temp, Wn[li], Ws[li], b2[li])
        li += 1
    h, coords_pad = _gcn_final(
        _segsum(src_p, dst_p, h), h, Wn[13], Ws[13], b2[13], We_pad, be_pad
    )
    return (h, coords_pad[:, :3])
